# Initial kernel scaffold; baseline (speedup 1.0000x reference)
import jax
import jax.numpy as jnp
from jax.experimental import pallas as pl


def kernel(pos_scores, neg_scores):
    raise NotImplementedError("write your pallas kernel here")


# speedup vs baseline: 22.7766x; 22.7766x over previous
import functools

import numpy as np
import jax
import jax.numpy as jnp
from jax import lax
from jax.experimental import pallas as pl
from jax.experimental.pallas import tpu as pltpu

P = 4096
K = 8192
N = 1000000
NROWS = 7840
NPAD = NROWS * 128

NCHEB = 16
_jj = np.arange(NCHEB) + 0.5
_NODES_X = np.cos(_jj * np.pi / NCHEB)
_CHEB_T = (2.0 / NCHEB) * np.cos(
    np.outer(np.arange(NCHEB), _jj) * np.pi / NCHEB)


def _softplus_sum(v, pos):
    d = v - pos
    return jnp.sum(jnp.maximum(d, 0.0) + jnp.log(1.0 + jnp.exp(-jnp.abs(d))))


def _tc_body(pos_ref, neg_ref, out_ref, ukey_ref):
    bits = lax.bitcast_convert_type(neg_ref[...], jnp.uint32)
    sign = bits >> jnp.uint32(31)
    ukey_ref[...] = jnp.where(sign == jnp.uint32(1), ~bits,
                              bits | jnp.uint32(0x80000000))

    def bs_body(i, prefix):
        cand = prefix | (jnp.uint32(1) << (jnp.uint32(31) - i.astype(jnp.uint32)))
        cnt = jnp.sum((ukey_ref[...] >= cand).astype(jnp.float32))
        return jnp.where(cnt >= float(K), cand, prefix)

    t_ukey = lax.fori_loop(0, 32, bs_body, jnp.uint32(0))
    tau_bits = jnp.where((t_ukey >> jnp.uint32(31)) == jnp.uint32(1),
                         t_ukey ^ jnp.uint32(0x80000000), ~t_ukey)
    tau = lax.bitcast_convert_type(tau_bits, jnp.float32)

    vmax = jnp.max(neg_ref[...])
    c0 = (tau + vmax) * 0.5 + 5e-4
    c1 = (vmax - tau) * 0.5 + 1e-3

    pos = pos_ref[...]
    gvals = [_softplus_sum(c0 + c1 * float(_NODES_X[j]), pos) for j in range(NCHEB)]
    g_tau = _softplus_sum(tau, pos)
    coef = [sum(float(_CHEB_T[kk, j]) * gvals[j] for j in range(NCHEB))
            for kk in range(NCHEB)]

    x = (neg_ref[...] - c0) / c1
    mask = ukey_ref[...] >= t_ukey
    b1 = jnp.zeros_like(x)
    b2 = jnp.zeros_like(x)
    for kk in range(NCHEB - 1, 0, -1):
        b0 = coef[kk] + (2.0 * x) * b1 - b2
        b2 = b1
        b1 = b0
    f = 0.5 * coef[0] + x * b1 - b2
    s_poly = jnp.sum(jnp.where(mask, f, 0.0))
    cnt_ge = jnp.sum(mask.astype(jnp.float32))
    loss_sum = s_poly + (float(K) - cnt_ge) * g_tau
    out_ref[0, 0] = loss_sum / float(P * K)


@functools.partial(jax.jit)
def kernel(pos_scores, neg_scores):
    pos2d = pos_scores.reshape(32, 128)
    pad = jnp.full((NPAD - N,), -jnp.inf, dtype=jnp.float32)
    neg2d = jnp.concatenate([neg_scores, pad]).reshape(NROWS, 128)
    out = pl.pallas_call(
        _tc_body,
        out_shape=jax.ShapeDtypeStruct((1, 1), jnp.float32),
        out_specs=pl.BlockSpec(memory_space=pltpu.SMEM),
        scratch_shapes=[pltpu.VMEM((NROWS, 128), jnp.uint32)],
    )(pos2d, neg2d)
    return out[0, 0]
